# baseline (speedup 1.0000x reference)
import functools

import jax
import jax.numpy as jnp
from jax import lax
from jax.experimental import pallas as pl
from jax.experimental.pallas import tpu as pltpu
from jax.experimental.pallas import tpu_sc as plsc

_NC = 2
_NS = 16
_NW = _NC * _NS
_L = 16

_D = 64
_BLK = 512
_GS = 128


def _sub_fold_body(sub_ref, wpT_ref, wc2T_ref, out_ref):
    tmp = jnp.dot(sub_ref[...], wpT_ref[...], preferred_element_type=jnp.float32)
    out_ref[...] = jnp.dot(tmp, wc2T_ref[...], preferred_element_type=jnp.float32)


def _cat_fold_body(cat_ref, wc1T_ref, wc2T_ref, bp_ref, bc_ref, out_ref):
    bias = bc_ref[...] + jnp.dot(bp_ref[...], wc2T_ref[...],
                                 preferred_element_type=jnp.float32)
    out_ref[...] = jnp.dot(cat_ref[...], wc1T_ref[...],
                           preferred_element_type=jnp.float32) + bias


def _fold_tables(cat_table, sub_table, W_proj, b_proj, W_comb, b_comb):
    wc1T = W_comb[:, :_D].T
    wc2T = W_comb[:, _D:].T
    wpT = W_proj.T
    n_sub, sub_dim = sub_table.shape
    rb = 2000
    sub_contrib = pl.pallas_call(
        _sub_fold_body,
        grid=(n_sub // rb,),
        in_specs=[
            pl.BlockSpec((rb, sub_dim), lambda i: (i, 0)),
            pl.BlockSpec(wpT.shape, lambda i: (0, 0)),
            pl.BlockSpec(wc2T.shape, lambda i: (0, 0)),
        ],
        out_specs=pl.BlockSpec((rb, _D), lambda i: (i, 0)),
        out_shape=jax.ShapeDtypeStruct((n_sub, _D), jnp.float32),
    )(sub_table, wpT, wc2T)
    cat_contrib = pl.pallas_call(
        _cat_fold_body,
        out_shape=jax.ShapeDtypeStruct((cat_table.shape[0], _D), jnp.float32),
    )(cat_table, wc1T, wc2T, b_proj.reshape(1, _D), b_comb.reshape(1, _D))
    return cat_contrib, sub_contrib


@functools.cache
def _make_sc_lookup(n_tokens):
    blk = 256
    ng = blk // _GS
    assert n_tokens % (_NW * 2 * blk) == 0
    per_w = n_tokens // _NW
    n_blk = per_w // blk
    mesh = plsc.VectorSubcoreMesh(core_axis_name="c", subcore_axis_name="s")

    @functools.partial(
        pl.kernel,
        out_type=jax.ShapeDtypeStruct((n_tokens, 2 * _D), jnp.float32),
        mesh=mesh,
        scratch_types=[
            pltpu.VMEM((blk,), jnp.int32), pltpu.VMEM((blk,), jnp.int32),
            pltpu.VMEM((blk,), jnp.int32), pltpu.VMEM((blk,), jnp.int32),
            pltpu.VMEM((blk, _D), jnp.float32), pltpu.VMEM((blk, _D), jnp.float32),
            pltpu.VMEM((blk, _D), jnp.float32), pltpu.VMEM((blk, _D), jnp.float32),
            pltpu.SemaphoreType.DMA, pltpu.SemaphoreType.DMA,
            pltpu.SemaphoreType.DMA, pltpu.SemaphoreType.DMA,
            pltpu.SemaphoreType.DMA, pltpu.SemaphoreType.DMA,
        ],
        compiler_params=pltpu.CompilerParams(use_tc_tiling_on_sc=False),
    )
    def sc_lookup(cat_hbm, sub_hbm, cid_hbm, sid_hbm, out_hbm,
                  cidx0, sidx0, cidx1, sidx1, crow0, srow0, crow1, srow1,
                  gsem0, gsem1, isem0, isem1, osem0, osem1):
        wid = lax.axis_index("s") * _NC + lax.axis_index("c")
        w_base = wid * per_w
        bufs = ((cidx0, sidx0, crow0, srow0, gsem0, isem0, osem0),
                (cidx1, sidx1, crow1, srow1, gsem1, isem1, osem1))

        def issue_idx(i, p):
            cidx, sidx, _, _, _, isem, _ = bufs[p]
            base = w_base + i * blk
            pltpu.async_copy(cid_hbm.at[pl.ds(base, blk)], cidx, isem)
            pltpu.async_copy(sid_hbm.at[pl.ds(base, blk)], sidx, isem)

        def wait_idx(i, p):
            cidx, sidx, _, _, _, isem, _ = bufs[p]
            base = w_base + i * blk
            pltpu.make_async_copy(cid_hbm.at[pl.ds(base, blk)], cidx, isem).wait()
            pltpu.make_async_copy(sid_hbm.at[pl.ds(base, blk)], sidx, isem).wait()

        def issue_gathers(p):
            cidx, sidx, crow, srow, gsem, _, _ = bufs[p]
            for g in range(ng):
                sl = pl.ds(g * _GS, _GS)
                pltpu.async_copy(cat_hbm.at[cidx.at[sl]], crow.at[sl], gsem)
                pltpu.async_copy(sub_hbm.at[sidx.at[sl]], srow.at[sl], gsem)

        def wait_gathers(p):
            cidx, sidx, crow, srow, gsem, _, _ = bufs[p]
            for g in range(ng):
                sl = pl.ds(g * _GS, _GS)
                pltpu.make_async_copy(cat_hbm.at[cidx.at[sl]], crow.at[sl], gsem).wait()
                pltpu.make_async_copy(sub_hbm.at[sidx.at[sl]], srow.at[sl], gsem).wait()

        def issue_store(i, p):
            _, _, crow, _, _, _, osem = bufs[p]
            base = w_base + i * blk
            pltpu.async_copy(crow, out_hbm.at[pl.ds(base, blk), pl.ds(0, _D)], osem)

        def wait_store(i, p):
            _, _, crow, _, _, _, osem = bufs[p]
            base = w_base + i * blk
            pltpu.make_async_copy(
                crow, out_hbm.at[pl.ds(base, blk), pl.ds(0, _D)], osem).wait()

        def do_add(p):
            _, _, crow, srow, _, _, _ = bufs[p]

            @plsc.parallel_loop(0, blk, 4)
            def _(t):
                for dt in range(4):
                    tt = t + dt
                    for k in range(_D // _L):
                        s2 = pl.ds(k * _L, _L)
                        crow[tt, s2] = crow[tt, s2] + srow[tt, s2]

        @pl.loop(0, n_blk)
        def _(i):
            base = w_base + i * blk
            pltpu.sync_copy(cid_hbm.at[pl.ds(base, blk)], cidx0)
            pltpu.sync_copy(sid_hbm.at[pl.ds(base, blk)], sidx0)
            for g in range(ng):
                sl = pl.ds(g * _GS, _GS)
                pltpu.async_copy(cat_hbm.at[cidx0.at[sl]], crow0.at[sl], gsem0)
            for g in range(ng):
                sl = pl.ds(g * _GS, _GS)
                pltpu.make_async_copy(cat_hbm.at[cidx0.at[sl]], crow0.at[sl], gsem0).wait()
            for g in range(ng):
                sl = pl.ds(g * _GS, _GS)
                pltpu.async_copy(sub_hbm.at[sidx0.at[sl]], crow0.at[sl], gsem0, add=True)
            for g in range(ng):
                sl = pl.ds(g * _GS, _GS)
                pltpu.make_async_copy(sub_hbm.at[sidx0.at[sl]], crow0.at[sl], gsem0).wait()
            pltpu.sync_copy(crow0, out_hbm.at[pl.ds(base, blk), pl.ds(0, _D)])

    return sc_lookup


def kernel(category_ids, subcategory_ids, cat_table, sub_table,
           W_proj, b_proj, W_comb, b_comb):
    cat_contrib, sub_contrib = _fold_tables(
        cat_table, sub_table, W_proj, b_proj, W_comb, b_comb)
    cid = category_ids.reshape(-1).astype(jnp.int32)
    sid = subcategory_ids.reshape(-1).astype(jnp.int32)
    out = _make_sc_lookup(cid.shape[0])(cat_contrib, sub_contrib, cid, sid)
    b, s = category_ids.shape
    return out.reshape(b, s, 2 * _D)[..., :_D]


# speedup vs baseline: 1.1765x; 1.1765x over previous
import functools

import jax
import jax.numpy as jnp
from jax import lax
from jax.experimental import pallas as pl
from jax.experimental.pallas import tpu as pltpu
from jax.experimental.pallas import tpu_sc as plsc

_NC = 2
_NS = 16
_NW = _NC * _NS
_L = 16

_D = 64
_BLK = 512
_GS = 128


def _sub_fold_body(sub_ref, wpT_ref, wc2T_ref, out_ref):
    tmp = jnp.dot(sub_ref[...], wpT_ref[...], preferred_element_type=jnp.float32)
    out_ref[...] = jnp.dot(tmp, wc2T_ref[...], preferred_element_type=jnp.float32)


def _cat_fold_body(cat_ref, wc1T_ref, wc2T_ref, bp_ref, bc_ref, out_ref):
    bias = bc_ref[...] + jnp.dot(bp_ref[...], wc2T_ref[...],
                                 preferred_element_type=jnp.float32)
    out_ref[...] = jnp.dot(cat_ref[...], wc1T_ref[...],
                           preferred_element_type=jnp.float32) + bias


def _fold_tables(cat_table, sub_table, W_proj, b_proj, W_comb, b_comb):
    wc1T = W_comb[:, :_D].T
    wc2T = W_comb[:, _D:].T
    wpT = W_proj.T
    n_sub, sub_dim = sub_table.shape
    rb = 2000
    sub_contrib = pl.pallas_call(
        _sub_fold_body,
        grid=(n_sub // rb,),
        in_specs=[
            pl.BlockSpec((rb, sub_dim), lambda i: (i, 0)),
            pl.BlockSpec(wpT.shape, lambda i: (0, 0)),
            pl.BlockSpec(wc2T.shape, lambda i: (0, 0)),
        ],
        out_specs=pl.BlockSpec((rb, _D), lambda i: (i, 0)),
        out_shape=jax.ShapeDtypeStruct((n_sub, _D), jnp.float32),
    )(sub_table, wpT, wc2T)
    cat_contrib = pl.pallas_call(
        _cat_fold_body,
        out_shape=jax.ShapeDtypeStruct((cat_table.shape[0], _D), jnp.float32),
    )(cat_table, wc1T, wc2T, b_proj.reshape(1, _D), b_comb.reshape(1, _D))
    return cat_contrib, sub_contrib


@functools.cache
def _make_sc_lookup(n_tokens):
    blk = 256
    ng = blk // _GS
    nbuf = 4
    assert n_tokens % (_NW * nbuf * blk) == 0
    per_w = n_tokens // _NW
    n_blk = per_w // blk
    n_out = -(-(n_blk + 2) // nbuf)
    mesh = plsc.VectorSubcoreMesh(core_axis_name="c", subcore_axis_name="s")

    @functools.partial(
        pl.kernel,
        out_type=jax.ShapeDtypeStruct((n_tokens, 2 * _D), jnp.float32),
        mesh=mesh,
        scratch_types=(
            [pltpu.VMEM((2, blk), jnp.int32) for _ in range(nbuf)]
            + [pltpu.VMEM((blk, _D), jnp.float32) for _ in range(nbuf)]
            + [pltpu.SemaphoreType.DMA] * (3 * nbuf)
        ),
        compiler_params=pltpu.CompilerParams(use_tc_tiling_on_sc=False),
    )
    def sc_lookup(cat_hbm, sub_hbm, ids_hbm, out_hbm, *scratch):
        idx = scratch[0:nbuf]
        row = scratch[nbuf:2 * nbuf]
        isem = scratch[2 * nbuf:3 * nbuf]
        gsem = scratch[3 * nbuf:4 * nbuf]
        osem = scratch[4 * nbuf:5 * nbuf]
        wid = lax.axis_index("s") * _NC + lax.axis_index("c")
        w_base = wid * per_w

        def issue_idx(i, s):
            base = w_base + i * blk
            pltpu.async_copy(ids_hbm.at[:, pl.ds(base, blk)], idx[s], isem[s])

        def wait_idx(i, s):
            base = w_base + i * blk
            pltpu.make_async_copy(ids_hbm.at[:, pl.ds(base, blk)], idx[s],
                                  isem[s]).wait()

        def issue_cat(s):
            for g in range(ng):
                sl = pl.ds(g * _GS, _GS)
                pltpu.async_copy(cat_hbm.at[idx[s].at[0, sl]], row[s].at[sl],
                                 gsem[s])

        def wait_cat(s):
            for g in range(ng):
                sl = pl.ds(g * _GS, _GS)
                pltpu.make_async_copy(cat_hbm.at[idx[s].at[0, sl]],
                                      row[s].at[sl], gsem[s]).wait()

        def issue_sub(s):
            for g in range(ng):
                sl = pl.ds(g * _GS, _GS)
                pltpu.async_copy(sub_hbm.at[idx[s].at[1, sl]], row[s].at[sl],
                                 gsem[s], add=True)

        def wait_sub(s):
            for g in range(ng):
                sl = pl.ds(g * _GS, _GS)
                pltpu.make_async_copy(sub_hbm.at[idx[s].at[1, sl]],
                                      row[s].at[sl], gsem[s]).wait()

        def issue_store(i, s):
            base = w_base + i * blk
            pltpu.async_copy(row[s], out_hbm.at[pl.ds(base, blk), pl.ds(0, _D)],
                             osem[s])

        def wait_store(i, s):
            base = w_base + i * blk
            pltpu.make_async_copy(row[s],
                                  out_hbm.at[pl.ds(base, blk), pl.ds(0, _D)],
                                  osem[s]).wait()

        for t in range(nbuf):
            pltpu.sync_copy(ids_hbm.at[:, pl.ds(w_base + t * blk, blk)], idx[t])

        @pl.loop(0, n_out)
        def _(o):
            for p in range(nbuf):
                j = nbuf * o + p

                @pl.when(j < n_blk)
                def _():
                    @pl.when(j >= nbuf)
                    def _():
                        wait_idx(j, p)

                    @pl.when(j >= nbuf)
                    def _():
                        wait_store(j - nbuf, p)

                    issue_cat(p)

                @pl.when((j >= 1) & (j - 1 < n_blk))
                def _():
                    s1 = (p - 1) % nbuf
                    wait_cat(s1)
                    issue_sub(s1)

                @pl.when((j >= 2) & (j - 2 < n_blk))
                def _():
                    s2 = (p - 2) % nbuf
                    wait_sub(s2)
                    issue_store(j - 2, s2)

                    @pl.when(j + 2 < n_blk)
                    def _():
                        issue_idx(j + 2, s2)

        for t in range(nbuf):
            i = n_blk - nbuf + t
            wait_store(i, i % nbuf)

    return sc_lookup


def kernel(category_ids, subcategory_ids, cat_table, sub_table,
           W_proj, b_proj, W_comb, b_comb):
    cat_contrib, sub_contrib = _fold_tables(
        cat_table, sub_table, W_proj, b_proj, W_comb, b_comb)
    cid = category_ids.reshape(-1).astype(jnp.int32)
    sid = subcategory_ids.reshape(-1).astype(jnp.int32)
    ids = jnp.stack([cid, sid])
    out = _make_sc_lookup(cid.shape[0])(cat_contrib, sub_contrib, ids)
    b, s = category_ids.shape
    return out.reshape(b, s, 2 * _D)[..., :_D]


# speedup vs baseline: 1.9011x; 1.6159x over previous
import functools

import jax
import jax.numpy as jnp
from jax import lax
from jax.experimental import pallas as pl
from jax.experimental.pallas import tpu as pltpu
from jax.experimental.pallas import tpu_sc as plsc

_NC = 2
_NS = 16
_NW = _NC * _NS

_D = 64
_GS = 128
_NCAT = 1024


def _sub_fold_body(sub2_ref, wbig_ref, out_ref):
    out_ref[...] = jnp.dot(sub2_ref[...], wbig_ref[...],
                           preferred_element_type=jnp.float32)


def _cat_fold_body(cat_ref, wc1T_ref, wc2T_ref, bp_ref, bc_ref, out_ref):
    bias = bc_ref[...] + jnp.dot(bp_ref[...], wc2T_ref[...],
                                 preferred_element_type=jnp.float32)
    out_ref[...] = jnp.dot(cat_ref[...], wc1T_ref[...],
                           preferred_element_type=jnp.float32) + bias


def _fold_tables(cat_table, sub_table, W_proj, b_proj, W_comb, b_comb):
    wc1T = W_comb[:, :_D].T
    wc2T = W_comb[:, _D:].T
    n_sub, sub_dim = sub_table.shape
    wb = W_proj.T @ wc2T
    wbig = jnp.zeros((2 * sub_dim, 4 * sub_dim), jnp.float32)
    wbig = wbig.at[:sub_dim, :2 * sub_dim].set(wb)
    wbig = wbig.at[sub_dim:, 2 * sub_dim:].set(wb)
    sub_pair = sub_table.reshape(n_sub // 2, 2 * sub_dim)
    rb = 5000
    sub_contrib = pl.pallas_call(
        _sub_fold_body,
        grid=(n_sub // 2 // rb,),
        in_specs=[
            pl.BlockSpec((rb, 2 * sub_dim), lambda i: (i, 0)),
            pl.BlockSpec(wbig.shape, lambda i: (0, 0)),
        ],
        out_specs=pl.BlockSpec((rb, 2 * _D), lambda i: (i, 0)),
        out_shape=jax.ShapeDtypeStruct((n_sub // 2, 2 * _D), jnp.float32),
    )(sub_pair, wbig)
    cat_padded = jnp.zeros((_NCAT, cat_table.shape[1]), jnp.float32)
    cat_padded = cat_padded.at[:cat_table.shape[0]].set(cat_table)
    cat_contrib = pl.pallas_call(
        _cat_fold_body,
        out_shape=jax.ShapeDtypeStruct((_NCAT, _D), jnp.float32),
    )(cat_padded, wc1T, wc2T, b_proj.reshape(1, _D), b_comb.reshape(1, _D))
    return cat_contrib, sub_contrib


@functools.cache
def _make_sc_lookup(n_tokens):
    blk = 256
    ng = blk // _GS
    nbuf = 4
    assert n_tokens % (_NW * nbuf * blk) == 0
    per_w = n_tokens // _NW
    n_blk = per_w // blk
    n_out = -(-(n_blk + 2) // nbuf)
    mesh = plsc.VectorSubcoreMesh(core_axis_name="c", subcore_axis_name="s")

    @functools.partial(
        pl.kernel,
        out_type=jax.ShapeDtypeStruct((n_tokens, 2 * _D), jnp.float32),
        mesh=mesh,
        scratch_types=(
            [pltpu.VMEM((2, blk), jnp.int32) for _ in range(nbuf)]
            + [pltpu.VMEM((blk, _D), jnp.float32) for _ in range(nbuf)]
            + [pltpu.VMEM_SHARED((_NCAT, _D), jnp.float32)]
            + [pltpu.SemaphoreType.DMA] * (3 * nbuf)
        ),
        compiler_params=pltpu.CompilerParams(use_tc_tiling_on_sc=False),
    )
    def sc_lookup(cat_hbm, sub_hbm, cid_hbm, sid_hbm, out_hbm, *scratch):
        idx = scratch[0:nbuf]
        row = scratch[nbuf:2 * nbuf]
        cat_sp = scratch[2 * nbuf]
        isem = scratch[2 * nbuf + 1:3 * nbuf + 1]
        gsem = scratch[3 * nbuf + 1:4 * nbuf + 1]
        osem = scratch[4 * nbuf + 1:5 * nbuf + 1]
        cid = lax.axis_index("c")
        sid = lax.axis_index("s")
        wid = sid * _NC + cid
        w_base = wid * per_w

        def issue_idx(i, s):
            base = w_base + i * blk
            pltpu.async_copy(cid_hbm.at[pl.ds(base, blk)], idx[s].at[0], isem[s])
            pltpu.async_copy(sid_hbm.at[pl.ds(base, blk)], idx[s].at[1], isem[s])

        def wait_idx(i, s):
            base = w_base + i * blk
            pltpu.make_async_copy(cid_hbm.at[pl.ds(base, blk)], idx[s].at[0],
                                  isem[s]).wait()
            pltpu.make_async_copy(sid_hbm.at[pl.ds(base, blk)], idx[s].at[1],
                                  isem[s]).wait()

        def issue_cat(s):
            for g in range(ng):
                sl = pl.ds(g * _GS, _GS)
                pltpu.async_copy(cat_sp.at[idx[s].at[0, sl]], row[s].at[sl],
                                 gsem[s])

        def wait_cat(s):
            for g in range(ng):
                sl = pl.ds(g * _GS, _GS)
                pltpu.make_async_copy(cat_sp.at[idx[s].at[0, sl]],
                                      row[s].at[sl], gsem[s]).wait()

        def issue_sub(s):
            for g in range(ng):
                sl = pl.ds(g * _GS, _GS)
                pltpu.async_copy(sub_hbm.at[idx[s].at[1, sl]], row[s].at[sl],
                                 gsem[s], add=True)

        def wait_sub(s):
            for g in range(ng):
                sl = pl.ds(g * _GS, _GS)
                pltpu.make_async_copy(sub_hbm.at[idx[s].at[1, sl]],
                                      row[s].at[sl], gsem[s]).wait()

        def issue_store(i, s):
            base = w_base + i * blk
            pltpu.async_copy(row[s], out_hbm.at[pl.ds(base, blk), pl.ds(0, _D)],
                             osem[s])

        def wait_store(i, s):
            base = w_base + i * blk
            pltpu.make_async_copy(row[s],
                                  out_hbm.at[pl.ds(base, blk), pl.ds(0, _D)],
                                  osem[s]).wait()

        rows_per = _NCAT // _NS
        pltpu.sync_copy(cat_hbm.at[pl.ds(sid * rows_per, rows_per)],
                        cat_sp.at[pl.ds(sid * rows_per, rows_per)])

        for t in range(nbuf):
            base = w_base + t * blk
            pltpu.sync_copy(cid_hbm.at[pl.ds(base, blk)], idx[t].at[0])
            pltpu.sync_copy(sid_hbm.at[pl.ds(base, blk)], idx[t].at[1])

        plsc.subcore_barrier()

        @pl.loop(0, n_out)
        def _(o):
            for p in range(nbuf):
                j = nbuf * o + p

                @pl.when(j < n_blk)
                def _():
                    @pl.when(j >= nbuf)
                    def _():
                        wait_idx(j, p)
                        wait_store(j - nbuf, p)

                    issue_cat(p)

                @pl.when((j >= 1) & (j - 1 < n_blk))
                def _():
                    s1 = (p - 1) % nbuf
                    wait_cat(s1)
                    issue_sub(s1)

                @pl.when((j >= 2) & (j - 2 < n_blk))
                def _():
                    s2 = (p - 2) % nbuf
                    wait_sub(s2)
                    issue_store(j - 2, s2)

                    @pl.when(j + 2 < n_blk)
                    def _():
                        issue_idx(j + 2, s2)

        for t in range(nbuf):
            i = n_blk - nbuf + t
            wait_store(i, i % nbuf)

    return sc_lookup


def kernel(category_ids, subcategory_ids, cat_table, sub_table,
           W_proj, b_proj, W_comb, b_comb):
    cat_contrib, sub_contrib = _fold_tables(
        cat_table, sub_table, W_proj, b_proj, W_comb, b_comb)
    cid = category_ids.reshape(-1).astype(jnp.int32)
    sid = subcategory_ids.reshape(-1).astype(jnp.int32)
    out = _make_sc_lookup(cid.shape[0])(
        cat_contrib, sub_contrib.reshape(-1, _D), cid, sid)
    b, s = category_ids.shape
    return out.reshape(b, s, 2 * _D)[..., :_D]


# speedup vs baseline: 1.9069x; 1.0030x over previous
import functools

import jax
import jax.numpy as jnp
from jax import lax
from jax.experimental import pallas as pl
from jax.experimental.pallas import tpu as pltpu
from jax.experimental.pallas import tpu_sc as plsc

_NC = 2
_NS = 16
_NW = _NC * _NS

_D = 64
_GS = 128
_NCAT = 1024


def _sub_fold_body(lo_ref, hi_ref, wb_ref, out_ref):
    lo = jnp.dot(lo_ref[...], wb_ref[...], preferred_element_type=jnp.float32)
    hi = jnp.dot(hi_ref[...], wb_ref[...], preferred_element_type=jnp.float32)
    out_ref[...] = jnp.concatenate([lo, hi], axis=1)


def _cat_fold_body(cat_ref, wc1T_ref, wc2T_ref, bp_ref, bc_ref, out_ref):
    bias = bc_ref[...] + jnp.dot(bp_ref[...], wc2T_ref[...],
                                 preferred_element_type=jnp.float32)
    out_ref[...] = jnp.dot(cat_ref[...], wc1T_ref[...],
                           preferred_element_type=jnp.float32) + bias


def _fold_tables(cat_table, sub_table, W_proj, b_proj, W_comb, b_comb):
    wc1T = W_comb[:, :_D].T
    wc2T = W_comb[:, _D:].T
    n_sub, sub_dim = sub_table.shape
    half = n_sub // 2
    wb = W_proj.T @ wc2T
    rb = 5000
    n_grid = half // rb
    sub_contrib = pl.pallas_call(
        _sub_fold_body,
        grid=(n_grid,),
        in_specs=[
            pl.BlockSpec((rb, sub_dim), lambda i: (i, 0)),
            pl.BlockSpec((rb, sub_dim), lambda i: (i + n_grid, 0)),
            pl.BlockSpec(wb.shape, lambda i: (0, 0)),
        ],
        out_specs=pl.BlockSpec((rb, 2 * _D), lambda i: (i, 0)),
        out_shape=jax.ShapeDtypeStruct((half, 2 * _D), jnp.float32),
    )(sub_table, sub_table, wb)
    cat_padded = jnp.zeros((_NCAT, cat_table.shape[1]), jnp.float32)
    cat_padded = cat_padded.at[:cat_table.shape[0]].set(cat_table)
    cat_contrib = pl.pallas_call(
        _cat_fold_body,
        out_shape=jax.ShapeDtypeStruct((_NCAT, _D), jnp.float32),
    )(cat_padded, wc1T, wc2T, b_proj.reshape(1, _D), b_comb.reshape(1, _D))
    return cat_contrib, sub_contrib


@functools.cache
def _make_sc_lookup(n_tokens):
    blk = 256
    ng = blk // _GS
    nbuf = 4
    assert n_tokens % (_NW * nbuf * blk) == 0
    per_w = n_tokens // _NW
    n_blk = per_w // blk
    n_out = -(-(n_blk + 2) // nbuf)
    mesh = plsc.VectorSubcoreMesh(core_axis_name="c", subcore_axis_name="s")

    @functools.partial(
        pl.kernel,
        out_type=jax.ShapeDtypeStruct((n_tokens, 2 * _D), jnp.float32),
        mesh=mesh,
        scratch_types=(
            [pltpu.VMEM((2, blk), jnp.int32) for _ in range(nbuf)]
            + [pltpu.VMEM((blk, _D), jnp.float32) for _ in range(nbuf)]
            + [pltpu.VMEM_SHARED((_NCAT, _D), jnp.float32)]
            + [pltpu.SemaphoreType.DMA] * (3 * nbuf)
        ),
        compiler_params=pltpu.CompilerParams(use_tc_tiling_on_sc=False),
    )
    def sc_lookup(cat_hbm, sub_hbm, cid_hbm, sid_hbm, out_hbm, *scratch):
        idx = scratch[0:nbuf]
        row = scratch[nbuf:2 * nbuf]
        cat_sp = scratch[2 * nbuf]
        isem = scratch[2 * nbuf + 1:3 * nbuf + 1]
        gsem = scratch[3 * nbuf + 1:4 * nbuf + 1]
        osem = scratch[4 * nbuf + 1:5 * nbuf + 1]
        cid = lax.axis_index("c")
        sid = lax.axis_index("s")
        wid = sid * _NC + cid
        w_base = wid * per_w

        def issue_idx(i, s):
            base = w_base + i * blk
            pltpu.async_copy(cid_hbm.at[pl.ds(base, blk)], idx[s].at[0], isem[s])
            pltpu.async_copy(sid_hbm.at[pl.ds(base, blk)], idx[s].at[1], isem[s])

        def wait_idx(i, s):
            base = w_base + i * blk
            pltpu.make_async_copy(cid_hbm.at[pl.ds(base, blk)], idx[s].at[0],
                                  isem[s]).wait()
            pltpu.make_async_copy(sid_hbm.at[pl.ds(base, blk)], idx[s].at[1],
                                  isem[s]).wait()

        def issue_cat(s):
            for g in range(ng):
                sl = pl.ds(g * _GS, _GS)
                pltpu.async_copy(cat_sp.at[idx[s].at[0, sl]], row[s].at[sl],
                                 gsem[s])

        def wait_cat(s):
            for g in range(ng):
                sl = pl.ds(g * _GS, _GS)
                pltpu.make_async_copy(cat_sp.at[idx[s].at[0, sl]],
                                      row[s].at[sl], gsem[s]).wait()

        def issue_sub(s):
            for g in range(ng):
                sl = pl.ds(g * _GS, _GS)
                pltpu.async_copy(sub_hbm.at[idx[s].at[1, sl]], row[s].at[sl],
                                 gsem[s], add=True)

        def wait_sub(s):
            for g in range(ng):
                sl = pl.ds(g * _GS, _GS)
                pltpu.make_async_copy(sub_hbm.at[idx[s].at[1, sl]],
                                      row[s].at[sl], gsem[s]).wait()

        def issue_store(i, s):
            base = w_base + i * blk
            pltpu.async_copy(row[s], out_hbm.at[pl.ds(base, blk), pl.ds(0, _D)],
                             osem[s])

        def wait_store(i, s):
            base = w_base + i * blk
            pltpu.make_async_copy(row[s],
                                  out_hbm.at[pl.ds(base, blk), pl.ds(0, _D)],
                                  osem[s]).wait()

        rows_per = _NCAT // _NS
        pltpu.sync_copy(cat_hbm.at[pl.ds(sid * rows_per, rows_per)],
                        cat_sp.at[pl.ds(sid * rows_per, rows_per)])

        for t in range(nbuf):
            base = w_base + t * blk
            pltpu.sync_copy(cid_hbm.at[pl.ds(base, blk)], idx[t].at[0])
            pltpu.sync_copy(sid_hbm.at[pl.ds(base, blk)], idx[t].at[1])

        plsc.subcore_barrier()

        @pl.loop(0, n_out)
        def _(o):
            for p in range(nbuf):
                j = nbuf * o + p

                @pl.when(j < n_blk)
                def _():
                    @pl.when(j >= nbuf)
                    def _():
                        wait_idx(j, p)
                        wait_store(j - nbuf, p)

                    issue_cat(p)

                @pl.when((j >= 1) & (j - 1 < n_blk))
                def _():
                    s1 = (p - 1) % nbuf
                    wait_cat(s1)
                    issue_sub(s1)

                @pl.when((j >= 2) & (j - 2 < n_blk))
                def _():
                    s2 = (p - 2) % nbuf
                    wait_sub(s2)
                    issue_store(j - 2, s2)

                    @pl.when(j + 2 < n_blk)
                    def _():
                        issue_idx(j + 2, s2)

        for t in range(nbuf):
            i = n_blk - nbuf + t
            wait_store(i, i % nbuf)

    return sc_lookup


def kernel(category_ids, subcategory_ids, cat_table, sub_table,
           W_proj, b_proj, W_comb, b_comb):
    cat_contrib, sub_contrib = _fold_tables(
        cat_table, sub_table, W_proj, b_proj, W_comb, b_comb)
    cid = category_ids.reshape(-1).astype(jnp.int32)
    sid = subcategory_ids.reshape(-1).astype(jnp.int32)
    half = sub_table.shape[0] // 2
    sid = jnp.where(sid < half, 2 * sid, 2 * (sid - half) + 1)
    out = _make_sc_lookup(cid.shape[0])(
        cat_contrib, sub_contrib.reshape(-1, _D), cid, sid)
    b, s = category_ids.shape
    return out.reshape(b, s, 2 * _D)[..., :_D]


# speedup vs baseline: 2.0117x; 1.0549x over previous
import functools

import jax
import jax.numpy as jnp
from jax import lax
from jax.experimental import pallas as pl
from jax.experimental.pallas import tpu as pltpu
from jax.experimental.pallas import tpu_sc as plsc

_NC = 2
_NS = 16
_NW = _NC * _NS

_D = 64
_GS = 128
_NCAT = 1024


def _sub_fold_body(subT_ref, wb_ref, out_ref, *, half, chunk):
    dn = (((0,), (0,)), ((), ()))
    for c in range(half // chunk):
        lo = lax.dot_general(subT_ref[:, c * chunk:(c + 1) * chunk],
                             wb_ref[...], dn, preferred_element_type=jnp.float32)
        hi = lax.dot_general(subT_ref[:, half + c * chunk:half + (c + 1) * chunk],
                             wb_ref[...], dn, preferred_element_type=jnp.float32)
        out_ref[c * chunk:(c + 1) * chunk, :] = jnp.concatenate([lo, hi], axis=1)


def _cat_fold_body(cat_ref, wc1T_ref, wc2T_ref, bp_ref, bc_ref, out_ref):
    bias = bc_ref[...] + jnp.dot(bp_ref[...], wc2T_ref[...],
                                 preferred_element_type=jnp.float32)
    out_ref[...] = jnp.dot(cat_ref[...], wc1T_ref[...],
                           preferred_element_type=jnp.float32) + bias


def _fold_tables(cat_table, sub_table, W_proj, b_proj, W_comb, b_comb):
    wc1T = W_comb[:, :_D].T
    wc2T = W_comb[:, _D:].T
    n_sub, sub_dim = sub_table.shape
    half = n_sub // 2
    wb = W_proj.T @ wc2T
    subT = sub_table.T
    sub_contrib = pl.pallas_call(
        functools.partial(_sub_fold_body, half=half, chunk=5000),
        out_shape=jax.ShapeDtypeStruct((half, 2 * _D), jnp.float32),
    )(subT, wb)
    cat_padded = jnp.zeros((_NCAT, cat_table.shape[1]), jnp.float32)
    cat_padded = cat_padded.at[:cat_table.shape[0]].set(cat_table)
    cat_contrib = pl.pallas_call(
        _cat_fold_body,
        out_shape=jax.ShapeDtypeStruct((_NCAT, _D), jnp.float32),
    )(cat_padded, wc1T, wc2T, b_proj.reshape(1, _D), b_comb.reshape(1, _D))
    return cat_contrib, sub_contrib


@functools.cache
def _make_sc_lookup(n_tokens):
    blk = 256
    ng = blk // _GS
    nbuf = 4
    assert n_tokens % (_NW * nbuf * blk) == 0
    per_w = n_tokens // _NW
    n_blk = per_w // blk
    n_out = -(-(n_blk + 2) // nbuf)
    mesh = plsc.VectorSubcoreMesh(core_axis_name="c", subcore_axis_name="s")

    @functools.partial(
        pl.kernel,
        out_type=jax.ShapeDtypeStruct((n_tokens, 2 * _D), jnp.float32),
        mesh=mesh,
        scratch_types=(
            [pltpu.VMEM((2, blk), jnp.int32) for _ in range(nbuf)]
            + [pltpu.VMEM((blk, _D), jnp.float32) for _ in range(nbuf)]
            + [pltpu.VMEM_SHARED((_NCAT, _D), jnp.float32)]
            + [pltpu.SemaphoreType.DMA] * (3 * nbuf)
        ),
        compiler_params=pltpu.CompilerParams(use_tc_tiling_on_sc=False),
    )
    def sc_lookup(cat_hbm, sub_hbm, cid_hbm, sid_hbm, out_hbm, *scratch):
        idx = scratch[0:nbuf]
        row = scratch[nbuf:2 * nbuf]
        cat_sp = scratch[2 * nbuf]
        isem = scratch[2 * nbuf + 1:3 * nbuf + 1]
        gsem = scratch[3 * nbuf + 1:4 * nbuf + 1]
        osem = scratch[4 * nbuf + 1:5 * nbuf + 1]
        cid = lax.axis_index("c")
        sid = lax.axis_index("s")
        wid = sid * _NC + cid
        w_base = wid * per_w

        def issue_idx(i, s):
            base = w_base + i * blk
            pltpu.async_copy(cid_hbm.at[pl.ds(base, blk)], idx[s].at[0], isem[s])
            pltpu.async_copy(sid_hbm.at[pl.ds(base, blk)], idx[s].at[1], isem[s])

        def wait_idx(i, s):
            base = w_base + i * blk
            pltpu.make_async_copy(cid_hbm.at[pl.ds(base, blk)], idx[s].at[0],
                                  isem[s]).wait()
            pltpu.make_async_copy(sid_hbm.at[pl.ds(base, blk)], idx[s].at[1],
                                  isem[s]).wait()

        def issue_cat(s):
            for g in range(ng):
                sl = pl.ds(g * _GS, _GS)
                pltpu.async_copy(cat_sp.at[idx[s].at[0, sl]], row[s].at[sl],
                                 gsem[s])

        def wait_cat(s):
            for g in range(ng):
                sl = pl.ds(g * _GS, _GS)
                pltpu.make_async_copy(cat_sp.at[idx[s].at[0, sl]],
                                      row[s].at[sl], gsem[s]).wait()

        def issue_sub(s):
            for g in range(ng):
                sl = pl.ds(g * _GS, _GS)
                pltpu.async_copy(sub_hbm.at[idx[s].at[1, sl]], row[s].at[sl],
                                 gsem[s], add=True)

        def wait_sub(s):
            for g in range(ng):
                sl = pl.ds(g * _GS, _GS)
                pltpu.make_async_copy(sub_hbm.at[idx[s].at[1, sl]],
                                      row[s].at[sl], gsem[s]).wait()

        def issue_store(i, s):
            base = w_base + i * blk
            pltpu.async_copy(row[s], out_hbm.at[pl.ds(base, blk), pl.ds(0, _D)],
                             osem[s])

        def wait_store(i, s):
            base = w_base + i * blk
            pltpu.make_async_copy(row[s],
                                  out_hbm.at[pl.ds(base, blk), pl.ds(0, _D)],
                                  osem[s]).wait()

        rows_per = _NCAT // _NS
        pltpu.sync_copy(cat_hbm.at[pl.ds(sid * rows_per, rows_per)],
                        cat_sp.at[pl.ds(sid * rows_per, rows_per)])

        for t in range(nbuf):
            base = w_base + t * blk
            pltpu.sync_copy(cid_hbm.at[pl.ds(base, blk)], idx[t].at[0])
            pltpu.sync_copy(sid_hbm.at[pl.ds(base, blk)], idx[t].at[1])

        plsc.subcore_barrier()

        @pl.loop(0, n_out)
        def _(o):
            for p in range(nbuf):
                j = nbuf * o + p

                @pl.when(j < n_blk)
                def _():
                    @pl.when(j >= nbuf)
                    def _():
                        wait_idx(j, p)
                        wait_store(j - nbuf, p)

                    issue_cat(p)

                @pl.when((j >= 1) & (j - 1 < n_blk))
                def _():
                    s1 = (p - 1) % nbuf
                    wait_cat(s1)
                    issue_sub(s1)

                @pl.when((j >= 2) & (j - 2 < n_blk))
                def _():
                    s2 = (p - 2) % nbuf
                    wait_sub(s2)
                    issue_store(j - 2, s2)

                    @pl.when(j + 2 < n_blk)
                    def _():
                        issue_idx(j + 2, s2)

        for t in range(nbuf):
            i = n_blk - nbuf + t
            wait_store(i, i % nbuf)

    return sc_lookup


def kernel(category_ids, subcategory_ids, cat_table, sub_table,
           W_proj, b_proj, W_comb, b_comb):
    cat_contrib, sub_contrib = _fold_tables(
        cat_table, sub_table, W_proj, b_proj, W_comb, b_comb)
    cid = category_ids.reshape(-1).astype(jnp.int32)
    sid = subcategory_ids.reshape(-1).astype(jnp.int32)
    half = sub_table.shape[0] // 2
    sid = jnp.where(sid < half, 2 * sid, 2 * (sid - half) + 1)
    out = _make_sc_lookup(cid.shape[0])(
        cat_contrib, sub_contrib.reshape(-1, _D), cid, sid)
    b, s = category_ids.shape
    return out.reshape(b, s, 2 * _D)[..., :_D]
